# baseline (speedup 1.0000x reference)
import functools

import jax
import jax.numpy as jnp
from jax import lax
from jax.experimental import pallas as pl
from jax.experimental.pallas import tpu as pltpu
from jax.experimental.pallas import tpu_sc as plsc

N = 10000
NP = 10240
D = 256
DH = 128
E = 160000
EP = 163840
PAD = EP - E
NSUB = 16
NCORE = 2
NPT = NP // NSUB
CH = 64
EPT = EP // NSUB
NCHUNK = EPT // CH
NQ = NCHUNK // 4
DEG_CW = 128
DEG_ROWS_W = EP // DEG_CW // (NCORE * NSUB)

_mesh = plsc.VectorSubcoreMesh(
    core_axis_name="c", subcore_axis_name="s",
    num_cores=NCORE, num_subcores=NSUB)

_f32 = jnp.float32


def _deg_body(dst2_hbm, deg0_hbm, deg1_hbm, dacc, idx_v, ones_v, zbuf):
    c = lax.axis_index("c")
    s = lax.axis_index("s")
    w = c * NSUB + s
    rsl = pl.ds(s * NPT, NPT)

    @pl.loop(0, NPT // 16)
    def _(i):
        zbuf[pl.ds(i * 16, 16)] = jnp.zeros((16,), _f32)

    @pl.loop(0, DEG_CW // 16)
    def _(i):
        ones_v[pl.ds(i * 16, 16)] = jnp.full((16,), 1.0, _f32)

    pltpu.sync_copy(zbuf, dacc.at[rsl])
    pltpu.sync_copy(dst2_hbm.at[pl.ds(w * DEG_ROWS_W, DEG_ROWS_W)], idx_v)
    plsc.subcore_barrier()

    @pl.loop(0, DEG_ROWS_W)
    def _(j):
        pltpu.sync_copy(ones_v, dacc.at[idx_v.at[j]], add=True)

    plsc.subcore_barrier()

    @pl.when(c == 0)
    def _():
        pltpu.sync_copy(dacc.at[rsl], deg0_hbm.at[rsl])

    @pl.when(c == 1)
    def _():
        pltpu.sync_copy(dacc.at[rsl], deg1_hbm.at[rsl])


def _deg_call(dst2):
    return pl.kernel(
        _deg_body,
        out_type=[jax.ShapeDtypeStruct((NP,), _f32),
                  jax.ShapeDtypeStruct((NP,), _f32)],
        mesh=_mesh,
        scratch_types=[
            pltpu.VMEM_SHARED((NP,), _f32),
            pltpu.VMEM((DEG_ROWS_W, DEG_CW), jnp.int32),
            pltpu.VMEM((DEG_CW,), _f32),
            pltpu.VMEM((NPT,), _f32),
        ],
    )(dst2)


def _agg_body(tlo_hbm, thi_hbm, src1_hbm, dst1_hbm, olo_hbm, ohi_hbm,
              acc, sbuf, dbuf, rows, semsi, semdi, semg, sems):
    c = lax.axis_index("c")
    s = lax.axis_index("s")
    rsl = pl.ds(s * NPT, NPT)
    ebase = s * EPT

    def echunk(k):
        return pl.ds(ebase + k * CH, CH)

    def rslice(b):
        return rows.at[pl.ds(b * CH, CH)]

    def sidx_load(k, b):
        pltpu.async_copy(src1_hbm.at[echunk(k)], sbuf.at[b], semsi.at[b])

    def sidx_wait(k, b):
        pltpu.make_async_copy(src1_hbm.at[echunk(k)], sbuf.at[b],
                              semsi.at[b]).wait()

    def didx_load(k, b):
        pltpu.async_copy(dst1_hbm.at[echunk(k)], dbuf.at[b], semdi.at[b])

    def didx_wait(k, b):
        pltpu.make_async_copy(dst1_hbm.at[echunk(k)], dbuf.at[b],
                              semdi.at[b]).wait()

    @pl.when(c == 0)
    def _():
        pltpu.sync_copy(tlo_hbm.at[rsl], acc.at[rsl])

    @pl.when(c == 1)
    def _():
        pltpu.sync_copy(thi_hbm.at[rsl], acc.at[rsl])

    for j in range(4):
        sidx_load(j, j)
    for j in range(2):
        didx_load(j, j)
    plsc.subcore_barrier()

    def run(tbl):
        def gather(k, b):
            pltpu.async_copy(tbl.at[sbuf.at[b]], rslice(b), semg.at[b])

        def gwait(b):
            pltpu.make_async_copy(tbl.at[sbuf.at[b]], rslice(b),
                                  semg.at[b]).wait()

        def scat(b):
            pltpu.async_copy(rslice(b), acc.at[dbuf.at[b]], sems.at[b],
                             add=True)

        def swait(b):
            pltpu.make_async_copy(rslice(b), acc.at[dbuf.at[b]],
                                  sems.at[b]).wait()

        sidx_wait(0, 0)
        gather(0, 0)
        sidx_wait(1, 1)
        gather(1, 1)

        def chunk_body(q, j):
            k = 4 * q + j
            b = j
            b2 = (j + 2) % 4
            gwait(b)

            @pl.when(q < NQ - 1)
            def _():
                sidx_load(k + 4, b)

            didx_wait(k, b)
            scat(b)

            if j >= 2:
                swait(b2)
            else:
                @pl.when(q >= 1)
                def _():
                    swait(b2)

            if j < 2:
                didx_load(k + 2, b2)
                sidx_wait(k + 2, b2)
                gather(k + 2, b2)
            else:
                @pl.when(q < NQ - 1)
                def _():
                    didx_load(k + 2, b2)
                    sidx_wait(k + 2, b2)
                    gather(k + 2, b2)

        @pl.loop(0, NQ)
        def _(q):
            for j in range(4):
                chunk_body(q, j)

        swait(2)
        swait(3)

    @pl.when(c == 0)
    def _():
        run(tlo_hbm)

    @pl.when(c == 1)
    def _():
        run(thi_hbm)

    plsc.subcore_barrier()

    @pl.when(c == 0)
    def _():
        pltpu.sync_copy(acc.at[rsl], olo_hbm.at[rsl])

    @pl.when(c == 1)
    def _():
        pltpu.sync_copy(acc.at[rsl], ohi_hbm.at[rsl])


def _agg_call(glo, ghi, src1, dst1):
    return pl.kernel(
        _agg_body,
        out_type=[jax.ShapeDtypeStruct((NP, DH), _f32),
                  jax.ShapeDtypeStruct((NP, DH), _f32)],
        mesh=_mesh,
        scratch_types=[
            pltpu.VMEM_SHARED((NP, DH), _f32),
            pltpu.VMEM((4, CH), jnp.int32),
            pltpu.VMEM((4, CH), jnp.int32),
            pltpu.VMEM((4 * CH, DH), _f32),
            pltpu.SemaphoreType.DMA((4,)),
            pltpu.SemaphoreType.DMA((4,)),
            pltpu.SemaphoreType.DMA((4,)),
            pltpu.SemaphoreType.DMA((4,)),
        ],
    )(glo, ghi, src1, dst1)


BM = 1024
_GRID = NP // BM
_DOT = functools.partial(jnp.dot, preferred_element_type=_f32,
                         precision=lax.Precision.HIGHEST)


def _mm1_body(x_ref, w_ref, h_ref):
    h_ref[...] = _DOT(x_ref[...], w_ref[...])


def _dinv_body(d0_ref, d1_ref, dv_ref):
    deg = d0_ref[...] + d1_ref[...] + 1.0
    dv_ref[...] = lax.rsqrt(deg)


def _prep_body(h_ref, dv_ref, glo_ref, ghi_ref):
    g = h_ref[...] * dv_ref[...]
    glo_ref[...] = g[:, :DH]
    ghi_ref[...] = g[:, DH:]


def _mid_body(alo_ref, ahi_ref, dv_ref, b1_ref, w2_ref, glo_ref, ghi_ref):
    a = jnp.concatenate([alo_ref[...], ahi_ref[...]], axis=1)
    z = jnp.maximum(a * dv_ref[...] + b1_ref[...], 0.0)
    h2 = _DOT(z, w2_ref[...])
    g2 = h2 * dv_ref[...]
    glo_ref[...] = g2[:, :DH]
    ghi_ref[...] = g2[:, DH:]


def _fin_body(alo_ref, ahi_ref, dv_ref, b2_ref, o_ref):
    a = jnp.concatenate([alo_ref[...], ahi_ref[...]], axis=1)
    o_ref[...] = a * dv_ref[...] + b2_ref[...]


def _row_spec(cols):
    return pl.BlockSpec((BM, cols), lambda i: (i, 0))


_FULL_W = pl.BlockSpec((D, D), lambda i: (0, 0))
_FULL_B = pl.BlockSpec((1, D), lambda i: (0, 0))
_DV = pl.BlockSpec((BM, 1), lambda i: (i, 0))


def _mm1_call(xp, W1):
    return pl.pallas_call(
        _mm1_body,
        grid=(_GRID,),
        in_specs=[_row_spec(D), _FULL_W],
        out_specs=_row_spec(D),
        out_shape=jax.ShapeDtypeStruct((NP, D), _f32),
    )(xp, W1)


def _dinv_call(deg0, deg1):
    return pl.pallas_call(
        _dinv_body,
        out_shape=jax.ShapeDtypeStruct((NP,), _f32),
    )(deg0, deg1)


def _prep_call(h1, dv):
    return pl.pallas_call(
        _prep_body,
        grid=(_GRID,),
        in_specs=[_row_spec(D), _DV],
        out_specs=[_row_spec(DH), _row_spec(DH)],
        out_shape=[jax.ShapeDtypeStruct((NP, DH), _f32),
                   jax.ShapeDtypeStruct((NP, DH), _f32)],
    )(h1, dv)


def _mid_call(alo, ahi, dv, b1, W2):
    return pl.pallas_call(
        _mid_body,
        grid=(_GRID,),
        in_specs=[_row_spec(DH), _row_spec(DH), _DV, _FULL_B, _FULL_W],
        out_specs=[_row_spec(DH), _row_spec(DH)],
        out_shape=[jax.ShapeDtypeStruct((NP, DH), _f32),
                   jax.ShapeDtypeStruct((NP, DH), _f32)],
    )(alo, ahi, dv, b1, W2)


def _fin_call(alo, ahi, dv, b2):
    return pl.pallas_call(
        _fin_body,
        grid=(_GRID,),
        in_specs=[_row_spec(DH), _row_spec(DH), _DV, _FULL_B],
        out_specs=_row_spec(D),
        out_shape=jax.ShapeDtypeStruct((NP, D), _f32),
    )(alo, ahi, dv, b2)


def kernel(src, dst, distances, x, edge_index, W1, b1, W2, b2):
    s_idx = edge_index[0].astype(jnp.int32)
    d_idx = edge_index[1].astype(jnp.int32)

    pad_idx = N + (jnp.arange(PAD, dtype=jnp.int32) % (NP - N))
    src1 = jnp.concatenate([s_idx, pad_idx])
    dst1 = jnp.concatenate([d_idx, pad_idx])
    dst2 = dst1.reshape(EP // DEG_CW, DEG_CW)

    xp = jnp.concatenate([x, jnp.zeros((NP - N, D), _f32)], axis=0)
    b1r = b1.reshape(1, D)
    b2r = b2.reshape(1, D)

    deg0, deg1 = _deg_call(dst2)
    h1 = _mm1_call(xp, W1)
    dv = _dinv_call(deg0, deg1).reshape(NP, 1)
    glo1, ghi1 = _prep_call(h1, dv)
    alo1, ahi1 = _agg_call(glo1, ghi1, src1, dst1)
    glo2, ghi2 = _mid_call(alo1, ahi1, dv, b1r, W2)
    alo2, ahi2 = _agg_call(glo2, ghi2, src1, dst1)
    out = _fin_call(alo2, ahi2, dv, b2r)
    return out[:N]


# speedup vs baseline: 1.1180x; 1.1180x over previous
import functools

import jax
import jax.numpy as jnp
from jax import lax
from jax.experimental import pallas as pl
from jax.experimental.pallas import tpu as pltpu
from jax.experimental.pallas import tpu_sc as plsc

N = 10000
NP = 10240
D = 256
DH = 128
E = 160000
EP = 163840
PAD = EP - E
NSUB = 16
NCORE = 2
NPT = NP // NSUB
CH = 128
EPT = EP // NSUB
NCHUNK = EPT // CH
NPAIR = NCHUNK // 2
DEG_CW = 128
DEG_ROWS_W = EP // DEG_CW // (NCORE * NSUB)

_mesh = plsc.VectorSubcoreMesh(
    core_axis_name="c", subcore_axis_name="s",
    num_cores=NCORE, num_subcores=NSUB)

_f32 = jnp.float32


def _deg_body(dst2_hbm, deg0_hbm, deg1_hbm, dacc, idx_v, ones_v, zbuf):
    c = lax.axis_index("c")
    s = lax.axis_index("s")
    w = c * NSUB + s
    rsl = pl.ds(s * NPT, NPT)

    @pl.loop(0, NPT // 16)
    def _(i):
        zbuf[pl.ds(i * 16, 16)] = jnp.zeros((16,), _f32)

    @pl.loop(0, DEG_CW // 16)
    def _(i):
        ones_v[pl.ds(i * 16, 16)] = jnp.full((16,), 1.0, _f32)

    pltpu.sync_copy(zbuf, dacc.at[rsl])
    pltpu.sync_copy(dst2_hbm.at[pl.ds(w * DEG_ROWS_W, DEG_ROWS_W)], idx_v)
    plsc.subcore_barrier()

    @pl.loop(0, DEG_ROWS_W)
    def _(j):
        pltpu.sync_copy(ones_v, dacc.at[idx_v.at[j]], add=True)

    plsc.subcore_barrier()

    @pl.when(c == 0)
    def _():
        pltpu.sync_copy(dacc.at[rsl], deg0_hbm.at[rsl])

    @pl.when(c == 1)
    def _():
        pltpu.sync_copy(dacc.at[rsl], deg1_hbm.at[rsl])


def _deg_call(dst2):
    return pl.kernel(
        _deg_body,
        out_type=[jax.ShapeDtypeStruct((NP,), _f32),
                  jax.ShapeDtypeStruct((NP,), _f32)],
        mesh=_mesh,
        scratch_types=[
            pltpu.VMEM_SHARED((NP,), _f32),
            pltpu.VMEM((DEG_ROWS_W, DEG_CW), jnp.int32),
            pltpu.VMEM((DEG_CW,), _f32),
            pltpu.VMEM((NPT,), _f32),
        ],
    )(dst2)


def _agg_body(tlo_hbm, thi_hbm, src2_hbm, dst2_hbm, olo_hbm, ohi_hbm,
              acc, sA, dA, sB, dB, rows0, rows1,
              semiA, semiB, semg0, semg1):
    c = lax.axis_index("c")
    s = lax.axis_index("s")
    rsl = pl.ds(s * NPT, NPT)
    rbase = s * NCHUNK

    @pl.when(c == 0)
    def _():
        pltpu.sync_copy(tlo_hbm.at[rsl], acc.at[rsl])

    @pl.when(c == 1)
    def _():
        pltpu.sync_copy(thi_hbm.at[rsl], acc.at[rsl])

    def load_idx(p, sbuf, dbuf, sem):
        rows = pl.ds(rbase + 2 * p, 2)
        pltpu.async_copy(src2_hbm.at[rows], sbuf, sem)
        pltpu.async_copy(dst2_hbm.at[rows], dbuf, sem)

    def wait_idx(p, sbuf, dbuf, sem):
        rows = pl.ds(rbase + 2 * p, 2)
        pltpu.make_async_copy(src2_hbm.at[rows], sbuf, sem).wait()
        pltpu.make_async_copy(dst2_hbm.at[rows], dbuf, sem).wait()

    load_idx(0, sA, dA, semiA)
    load_idx(1, sB, dB, semiB)
    plsc.subcore_barrier()

    def run(tbl):
        def gather(sbuf, half, rows, sem):
            pltpu.async_copy(tbl.at[sbuf.at[half]], rows, sem)

        def gwait(rows, sem):
            pltpu.make_async_copy(tbl.at[sA.at[0]], rows, sem).wait()

        wait_idx(0, sA, dA, semiA)
        gather(sA, 0, rows0, semg0)
        gather(sA, 1, rows1, semg1)

        def pair_body(p, sS, dS, semiS, sT, dT, semiT):
            gwait(rows0, semg0)
            pltpu.sync_copy(rows0, acc.at[dS.at[0]], add=True)

            @pl.when(p + 1 < NPAIR)
            def _():
                wait_idx(p + 1, sT, dT, semiT)
                gather(sT, 0, rows0, semg0)

            gwait(rows1, semg1)
            pltpu.sync_copy(rows1, acc.at[dS.at[1]], add=True)

            @pl.when(p + 2 < NPAIR)
            def _():
                load_idx(p + 2, sS, dS, semiS)

            @pl.when(p + 1 < NPAIR)
            def _():
                gather(sT, 1, rows1, semg1)

        @pl.loop(0, NPAIR // 2)
        def _(q):
            pair_body(2 * q, sA, dA, semiA, sB, dB, semiB)
            pair_body(2 * q + 1, sB, dB, semiB, sA, dA, semiA)

    @pl.when(c == 0)
    def _():
        run(tlo_hbm)

    @pl.when(c == 1)
    def _():
        run(thi_hbm)

    plsc.subcore_barrier()

    @pl.when(c == 0)
    def _():
        pltpu.sync_copy(acc.at[rsl], olo_hbm.at[rsl])

    @pl.when(c == 1)
    def _():
        pltpu.sync_copy(acc.at[rsl], ohi_hbm.at[rsl])


def _agg_call(glo, ghi, src2, dst2):
    return pl.kernel(
        _agg_body,
        out_type=[jax.ShapeDtypeStruct((NP, DH), _f32),
                  jax.ShapeDtypeStruct((NP, DH), _f32)],
        mesh=_mesh,
        scratch_types=[
            pltpu.VMEM_SHARED((NP, DH), _f32),
            pltpu.VMEM((2, CH), jnp.int32),
            pltpu.VMEM((2, CH), jnp.int32),
            pltpu.VMEM((2, CH), jnp.int32),
            pltpu.VMEM((2, CH), jnp.int32),
            pltpu.VMEM((CH, DH), _f32),
            pltpu.VMEM((CH, DH), _f32),
            pltpu.SemaphoreType.DMA,
            pltpu.SemaphoreType.DMA,
            pltpu.SemaphoreType.DMA,
            pltpu.SemaphoreType.DMA,
        ],
    )(glo, ghi, src2, dst2)


BM = 1024
_GRID = NP // BM
_DOT = functools.partial(jnp.dot, preferred_element_type=_f32,
                         precision=lax.Precision.HIGHEST)


def _mmprep_body(x_ref, w_ref, d0_ref, d1_ref, glo_ref, ghi_ref, dv_ref):
    dv = lax.rsqrt(d0_ref[...] + d1_ref[...] + 1.0)
    g = _DOT(x_ref[...], w_ref[...]) * dv
    glo_ref[...] = g[:, :DH]
    ghi_ref[...] = g[:, DH:]
    dv_ref[...] = dv


def _mid_body(alo_ref, ahi_ref, dv_ref, b1_ref, w2_ref, glo_ref, ghi_ref):
    a = jnp.concatenate([alo_ref[...], ahi_ref[...]], axis=1)
    z = jnp.maximum(a * dv_ref[...] + b1_ref[...], 0.0)
    h2 = _DOT(z, w2_ref[...])
    g2 = h2 * dv_ref[...]
    glo_ref[...] = g2[:, :DH]
    ghi_ref[...] = g2[:, DH:]


def _fin_body(alo_ref, ahi_ref, dv_ref, b2_ref, o_ref):
    a = jnp.concatenate([alo_ref[...], ahi_ref[...]], axis=1)
    o_ref[...] = a * dv_ref[...] + b2_ref[...]


def _row_spec(cols):
    return pl.BlockSpec((BM, cols), lambda i: (i, 0))


_FULL_W = pl.BlockSpec((D, D), lambda i: (0, 0))
_FULL_B = pl.BlockSpec((1, D), lambda i: (0, 0))
_DV = pl.BlockSpec((BM, 1), lambda i: (i, 0))


def _mmprep_call(xp, W1, deg0c, deg1c):
    return pl.pallas_call(
        _mmprep_body,
        grid=(_GRID,),
        in_specs=[_row_spec(D), _FULL_W, _DV, _DV],
        out_specs=[_row_spec(DH), _row_spec(DH), _DV],
        out_shape=[jax.ShapeDtypeStruct((NP, DH), _f32),
                   jax.ShapeDtypeStruct((NP, DH), _f32),
                   jax.ShapeDtypeStruct((NP, 1), _f32)],
    )(xp, W1, deg0c, deg1c)


def _mid_call(alo, ahi, dv, b1, W2):
    return pl.pallas_call(
        _mid_body,
        grid=(_GRID,),
        in_specs=[_row_spec(DH), _row_spec(DH), _DV, _FULL_B, _FULL_W],
        out_specs=[_row_spec(DH), _row_spec(DH)],
        out_shape=[jax.ShapeDtypeStruct((NP, DH), _f32),
                   jax.ShapeDtypeStruct((NP, DH), _f32)],
    )(alo, ahi, dv, b1, W2)


def _fin_call(alo, ahi, dv, b2):
    return pl.pallas_call(
        _fin_body,
        grid=(_GRID,),
        in_specs=[_row_spec(DH), _row_spec(DH), _DV, _FULL_B],
        out_specs=_row_spec(D),
        out_shape=jax.ShapeDtypeStruct((NP, D), _f32),
    )(alo, ahi, dv, b2)


def kernel(src, dst, distances, x, edge_index, W1, b1, W2, b2):
    s_idx = edge_index[0].astype(jnp.int32)
    d_idx = edge_index[1].astype(jnp.int32)

    pad_idx = N + (jnp.arange(PAD, dtype=jnp.int32) % (NP - N))
    src2 = jnp.concatenate([s_idx, pad_idx]).reshape(EP // CH, CH)
    dst2 = jnp.concatenate([d_idx, pad_idx]).reshape(EP // CH, CH)

    xp = jnp.concatenate([x, jnp.zeros((NP - N, D), _f32)], axis=0)
    b1r = b1.reshape(1, D)
    b2r = b2.reshape(1, D)

    deg0, deg1 = _deg_call(dst2)
    glo1, ghi1, dv = _mmprep_call(xp, W1, deg0.reshape(NP, 1),
                                  deg1.reshape(NP, 1))
    alo1, ahi1 = _agg_call(glo1, ghi1, src2, dst2)
    glo2, ghi2 = _mid_call(alo1, ahi1, dv, b1r, W2)
    alo2, ahi2 = _agg_call(glo2, ghi2, src2, dst2)
    out = _fin_call(alo2, ahi2, dv, b2r)
    return out[:N]


# speedup vs baseline: 1.1482x; 1.0270x over previous
import functools

import jax
import jax.numpy as jnp
from jax import lax
from jax.experimental import pallas as pl
from jax.experimental.pallas import tpu as pltpu
from jax.experimental.pallas import tpu_sc as plsc

N = 10000
NP = 10240
D = 256
DH = 128
E = 160000
EP = 163840
PAD = EP - E
NSUB = 16
NCORE = 2
NPT = NP // NSUB
CH = 128
EPT = EP // NSUB
NCHUNK = EPT // CH
NPAIR = NCHUNK // 2
DEG_CW = 128
DEG_ROWS_W = EP // DEG_CW // (NCORE * NSUB)

_mesh = plsc.VectorSubcoreMesh(
    core_axis_name="c", subcore_axis_name="s",
    num_cores=NCORE, num_subcores=NSUB)

_f32 = jnp.float32


def _deg_body(dst2_hbm, deg0_hbm, deg1_hbm, dacc, idx_v, ones_v, zbuf):
    c = lax.axis_index("c")
    s = lax.axis_index("s")
    w = c * NSUB + s
    rsl = pl.ds(s * NPT, NPT)

    @pl.loop(0, NPT // 16)
    def _(i):
        zbuf[pl.ds(i * 16, 16)] = jnp.zeros((16,), _f32)

    @pl.loop(0, DEG_CW // 16)
    def _(i):
        ones_v[pl.ds(i * 16, 16)] = jnp.full((16,), 1.0, _f32)

    pltpu.sync_copy(zbuf, dacc.at[rsl])
    pltpu.sync_copy(dst2_hbm.at[pl.ds(w * DEG_ROWS_W, DEG_ROWS_W)], idx_v)
    plsc.subcore_barrier()

    @pl.loop(0, DEG_ROWS_W)
    def _(j):
        pltpu.sync_copy(ones_v, dacc.at[idx_v.at[j]], add=True)

    plsc.subcore_barrier()

    @pl.when(c == 0)
    def _():
        pltpu.sync_copy(dacc.at[rsl], deg0_hbm.at[rsl])

    @pl.when(c == 1)
    def _():
        pltpu.sync_copy(dacc.at[rsl], deg1_hbm.at[rsl])


def _deg_call(dst2):
    return pl.kernel(
        _deg_body,
        out_type=[jax.ShapeDtypeStruct((NP,), _f32),
                  jax.ShapeDtypeStruct((NP,), _f32)],
        mesh=_mesh,
        scratch_types=[
            pltpu.VMEM_SHARED((NP,), _f32),
            pltpu.VMEM((DEG_ROWS_W, DEG_CW), jnp.int32),
            pltpu.VMEM((DEG_CW,), _f32),
            pltpu.VMEM((NPT,), _f32),
        ],
    )(dst2)


def _agg_body(tlo_hbm, thi_hbm, src2_hbm, dst2_hbm, olo_hbm, ohi_hbm,
              acc, sA, dA, sB, dB, rows0, rows1,
              semiA, semiB, semg0, semg1):
    c = lax.axis_index("c")
    s = lax.axis_index("s")
    rsl = pl.ds(s * NPT, NPT)
    rbase = s * NCHUNK

    @pl.when(c == 0)
    def _():
        pltpu.sync_copy(tlo_hbm.at[rsl], acc.at[rsl])

    @pl.when(c == 1)
    def _():
        pltpu.sync_copy(thi_hbm.at[rsl], acc.at[rsl])

    def load_idx(p, sbuf, dbuf, sem):
        rows = pl.ds(rbase + 2 * p, 2)
        pltpu.async_copy(src2_hbm.at[rows], sbuf, sem)
        pltpu.async_copy(dst2_hbm.at[rows], dbuf, sem)

    def wait_idx(p, sbuf, dbuf, sem):
        rows = pl.ds(rbase + 2 * p, 2)
        pltpu.make_async_copy(src2_hbm.at[rows], sbuf, sem).wait()
        pltpu.make_async_copy(dst2_hbm.at[rows], dbuf, sem).wait()

    load_idx(0, sA, dA, semiA)
    load_idx(1, sB, dB, semiB)
    plsc.subcore_barrier()

    def run(tbl):
        def gather(sbuf, half, rows, sem):
            pltpu.async_copy(tbl.at[sbuf.at[half]], rows, sem)

        def gwait(rows, sem):
            pltpu.make_async_copy(tbl.at[sA.at[0]], rows, sem).wait()

        wait_idx(0, sA, dA, semiA)
        gather(sA, 0, rows0, semg0)
        gather(sA, 1, rows1, semg1)

        def pair_body(p, sS, dS, semiS, sT, dT, semiT):
            gwait(rows0, semg0)
            pltpu.sync_copy(rows0, acc.at[dS.at[0]], add=True)

            @pl.when(p + 1 < NPAIR)
            def _():
                wait_idx(p + 1, sT, dT, semiT)
                gather(sT, 0, rows0, semg0)

            gwait(rows1, semg1)
            pltpu.sync_copy(rows1, acc.at[dS.at[1]], add=True)

            @pl.when(p + 2 < NPAIR)
            def _():
                load_idx(p + 2, sS, dS, semiS)

            @pl.when(p + 1 < NPAIR)
            def _():
                gather(sT, 1, rows1, semg1)

        @pl.loop(0, NPAIR // 2)
        def _(q):
            pair_body(2 * q, sA, dA, semiA, sB, dB, semiB)
            pair_body(2 * q + 1, sB, dB, semiB, sA, dA, semiA)

    @pl.when(c == 0)
    def _():
        run(tlo_hbm)

    @pl.when(c == 1)
    def _():
        run(thi_hbm)

    plsc.subcore_barrier()

    @pl.when(c == 0)
    def _():
        pltpu.sync_copy(acc.at[rsl], olo_hbm.at[rsl])

    @pl.when(c == 1)
    def _():
        pltpu.sync_copy(acc.at[rsl], ohi_hbm.at[rsl])


def _agg_call(glo, ghi, src2, dst2):
    return pl.kernel(
        _agg_body,
        out_type=[jax.ShapeDtypeStruct((NP, DH), _f32),
                  jax.ShapeDtypeStruct((NP, DH), _f32)],
        mesh=_mesh,
        scratch_types=[
            pltpu.VMEM_SHARED((NP, DH), _f32),
            pltpu.VMEM((2, CH), jnp.int32),
            pltpu.VMEM((2, CH), jnp.int32),
            pltpu.VMEM((2, CH), jnp.int32),
            pltpu.VMEM((2, CH), jnp.int32),
            pltpu.VMEM((CH, DH), _f32),
            pltpu.VMEM((CH, DH), _f32),
            pltpu.SemaphoreType.DMA,
            pltpu.SemaphoreType.DMA,
            pltpu.SemaphoreType.DMA,
            pltpu.SemaphoreType.DMA,
        ],
    )(glo, ghi, src2, dst2)


BM = 1000
_GRID = N // BM
_DOT = functools.partial(jnp.dot, preferred_element_type=_f32,
                         precision=lax.Precision.HIGHEST)


def _mmprep_body(x_ref, w_ref, d0_ref, d1_ref, glo_ref, ghi_ref, dv_ref):
    dv = lax.rsqrt(d0_ref[...] + d1_ref[...] + 1.0)
    g = _DOT(x_ref[...], w_ref[...]) * dv
    glo_ref[...] = g[:, :DH]
    ghi_ref[...] = g[:, DH:]
    dv_ref[...] = dv


def _mid_body(alo_ref, ahi_ref, dv_ref, b1_ref, w2_ref, glo_ref, ghi_ref):
    a = jnp.concatenate([alo_ref[...], ahi_ref[...]], axis=1)
    z = jnp.maximum(a * dv_ref[...] + b1_ref[...], 0.0)
    h2 = _DOT(z, w2_ref[...])
    g2 = h2 * dv_ref[...]
    glo_ref[...] = g2[:, :DH]
    ghi_ref[...] = g2[:, DH:]


def _fin_body(alo_ref, ahi_ref, dv_ref, b2_ref, o_ref):
    a = jnp.concatenate([alo_ref[...], ahi_ref[...]], axis=1)
    o_ref[...] = a * dv_ref[...] + b2_ref[...]


def _row_spec(cols):
    return pl.BlockSpec((BM, cols), lambda i: (i, 0))


_FULL_W = pl.BlockSpec((D, D), lambda i: (0, 0))
_FULL_B = pl.BlockSpec((1, D), lambda i: (0, 0))
_DV = pl.BlockSpec((BM, 1), lambda i: (i, 0))


def _mmprep_call(x, W1, deg0c, deg1c):
    return pl.pallas_call(
        _mmprep_body,
        grid=(_GRID,),
        in_specs=[_row_spec(D), _FULL_W, _DV, _DV],
        out_specs=[_row_spec(DH), _row_spec(DH), _DV],
        out_shape=[jax.ShapeDtypeStruct((NP, DH), _f32),
                   jax.ShapeDtypeStruct((NP, DH), _f32),
                   jax.ShapeDtypeStruct((NP, 1), _f32)],
    )(x, W1, deg0c, deg1c)


def _mid_call(alo, ahi, dv, b1, W2):
    return pl.pallas_call(
        _mid_body,
        grid=(_GRID,),
        in_specs=[_row_spec(DH), _row_spec(DH), _DV, _FULL_B, _FULL_W],
        out_specs=[_row_spec(DH), _row_spec(DH)],
        out_shape=[jax.ShapeDtypeStruct((NP, DH), _f32),
                   jax.ShapeDtypeStruct((NP, DH), _f32)],
    )(alo, ahi, dv, b1, W2)


def _fin_call(alo, ahi, dv, b2):
    return pl.pallas_call(
        _fin_body,
        grid=(_GRID,),
        in_specs=[_row_spec(DH), _row_spec(DH), _DV, _FULL_B],
        out_specs=_row_spec(D),
        out_shape=jax.ShapeDtypeStruct((N, D), _f32),
    )(alo, ahi, dv, b2)


def kernel(src, dst, distances, x, edge_index, W1, b1, W2, b2):
    s_idx = edge_index[0].astype(jnp.int32)
    d_idx = edge_index[1].astype(jnp.int32)

    pad_cyc = jnp.arange(PAD, dtype=jnp.int32) % (NP - N)
    src2 = jnp.concatenate([s_idx, pad_cyc]).reshape(EP // CH, CH)
    dst2 = jnp.concatenate([d_idx, N + pad_cyc]).reshape(EP // CH, CH)

    b1r = b1.reshape(1, D)
    b2r = b2.reshape(1, D)

    deg0, deg1 = _deg_call(dst2)
    glo1, ghi1, dv = _mmprep_call(x, W1, deg0.reshape(NP, 1),
                                  deg1.reshape(NP, 1))
    alo1, ahi1 = _agg_call(glo1, ghi1, src2, dst2)
    glo2, ghi2 = _mid_call(alo1, ahi1, dv, b1r, W2)
    alo2, ahi2 = _agg_call(glo2, ghi2, src2, dst2)
    return _fin_call(alo2, ahi2, dv, b2r)


# speedup vs baseline: 1.1940x; 1.0399x over previous
import functools

import jax
import jax.numpy as jnp
from jax import lax
from jax.experimental import pallas as pl
from jax.experimental.pallas import tpu as pltpu
from jax.experimental.pallas import tpu_sc as plsc

N = 10000
NP = 10240
D = 256
DH = 128
E = 160000
EP = 163840
PAD = EP - E
NSUB = 16
NCORE = 2
NPT = NP // NSUB
CH = 128
EPT = EP // NSUB
NCHUNK = EPT // CH
NPAIR = NCHUNK // 2
DEG_CW = 128
DEG_ROWS_W = EP // DEG_CW // (NCORE * NSUB)

_mesh = plsc.VectorSubcoreMesh(
    core_axis_name="c", subcore_axis_name="s",
    num_cores=NCORE, num_subcores=NSUB)

_f32 = jnp.float32


def _deg_body(dst2_hbm, deg0_hbm, deg1_hbm, dacc, idx_v, ones_v, zbuf):
    c = lax.axis_index("c")
    s = lax.axis_index("s")
    w = c * NSUB + s
    rsl = pl.ds(s * NPT, NPT)

    @pl.loop(0, NPT // 16)
    def _(i):
        zbuf[pl.ds(i * 16, 16)] = jnp.zeros((16,), _f32)

    @pl.loop(0, DEG_CW // 16)
    def _(i):
        ones_v[pl.ds(i * 16, 16)] = jnp.full((16,), 1.0, _f32)

    pltpu.sync_copy(zbuf, dacc.at[rsl])
    pltpu.sync_copy(dst2_hbm.at[pl.ds(w * DEG_ROWS_W, DEG_ROWS_W)], idx_v)
    plsc.subcore_barrier()

    @pl.loop(0, DEG_ROWS_W)
    def _(j):
        pltpu.sync_copy(ones_v, dacc.at[idx_v.at[j]], add=True)

    plsc.subcore_barrier()

    @pl.when(c == 0)
    def _():
        pltpu.sync_copy(dacc.at[rsl], deg0_hbm.at[rsl])

    @pl.when(c == 1)
    def _():
        pltpu.sync_copy(dacc.at[rsl], deg1_hbm.at[rsl])


def _deg_call(dst2):
    return pl.kernel(
        _deg_body,
        out_type=[jax.ShapeDtypeStruct((NP,), _f32),
                  jax.ShapeDtypeStruct((NP,), _f32)],
        mesh=_mesh,
        scratch_types=[
            pltpu.VMEM_SHARED((NP,), _f32),
            pltpu.VMEM((DEG_ROWS_W, DEG_CW), jnp.int32),
            pltpu.VMEM((DEG_CW,), _f32),
            pltpu.VMEM((NPT,), _f32),
        ],
    )(dst2)


def _agg_body(tlo_hbm, thi_hbm, src2_hbm, dst2_hbm, olo_hbm, ohi_hbm,
              acc, sA, dA, sB, dB, rows0, rows1,
              semiA, semiB, semg0, semg1):
    c = lax.axis_index("c")
    s = lax.axis_index("s")
    rsl = pl.ds(s * NPT, NPT)
    rbase = s * NCHUNK

    @pl.when(c == 0)
    def _():
        pltpu.sync_copy(tlo_hbm.at[rsl], acc.at[rsl])

    @pl.when(c == 1)
    def _():
        pltpu.sync_copy(thi_hbm.at[rsl], acc.at[rsl])

    def load_idx(p, sbuf, dbuf, sem):
        rows = pl.ds(rbase + 2 * p, 2)
        pltpu.async_copy(src2_hbm.at[rows], sbuf, sem)
        pltpu.async_copy(dst2_hbm.at[rows], dbuf, sem)

    def wait_idx(p, sbuf, dbuf, sem):
        rows = pl.ds(rbase + 2 * p, 2)
        pltpu.make_async_copy(src2_hbm.at[rows], sbuf, sem).wait()
        pltpu.make_async_copy(dst2_hbm.at[rows], dbuf, sem).wait()

    load_idx(0, sA, dA, semiA)
    load_idx(1, sB, dB, semiB)
    plsc.subcore_barrier()

    def run(tbl):
        def gather(sbuf, half, rows, sem):
            pltpu.async_copy(tbl.at[sbuf.at[half]], rows, sem)

        def gwait(rows, sem):
            pltpu.make_async_copy(tbl.at[sA.at[0]], rows, sem).wait()

        wait_idx(0, sA, dA, semiA)
        gather(sA, 0, rows0, semg0)
        gather(sA, 1, rows1, semg1)

        def pair_body(p, sS, dS, semiS, sT, dT, semiT):
            gwait(rows0, semg0)
            pltpu.sync_copy(rows0, acc.at[dS.at[0]], add=True)

            @pl.when(p + 1 < NPAIR)
            def _():
                wait_idx(p + 1, sT, dT, semiT)
                gather(sT, 0, rows0, semg0)

            gwait(rows1, semg1)
            pltpu.sync_copy(rows1, acc.at[dS.at[1]], add=True)

            @pl.when(p + 2 < NPAIR)
            def _():
                load_idx(p + 2, sS, dS, semiS)

            @pl.when(p + 1 < NPAIR)
            def _():
                gather(sT, 1, rows1, semg1)

        @pl.loop(0, NPAIR // 2)
        def _(q):
            pair_body(2 * q, sA, dA, semiA, sB, dB, semiB)
            pair_body(2 * q + 1, sB, dB, semiB, sA, dA, semiA)

    @pl.when(c == 0)
    def _():
        run(tlo_hbm)

    @pl.when(c == 1)
    def _():
        run(thi_hbm)

    plsc.subcore_barrier()

    @pl.when(c == 0)
    def _():
        pltpu.sync_copy(acc.at[rsl], olo_hbm.at[rsl])

    @pl.when(c == 1)
    def _():
        pltpu.sync_copy(acc.at[rsl], ohi_hbm.at[rsl])


def _agg_call(glo, ghi, src2, dst2):
    return pl.kernel(
        _agg_body,
        out_type=[jax.ShapeDtypeStruct((NP, DH), _f32),
                  jax.ShapeDtypeStruct((NP, DH), _f32)],
        mesh=_mesh,
        scratch_types=[
            pltpu.VMEM_SHARED((NP, DH), _f32),
            pltpu.VMEM((2, CH), jnp.int32),
            pltpu.VMEM((2, CH), jnp.int32),
            pltpu.VMEM((2, CH), jnp.int32),
            pltpu.VMEM((2, CH), jnp.int32),
            pltpu.VMEM((CH, DH), _f32),
            pltpu.VMEM((CH, DH), _f32),
            pltpu.SemaphoreType.DMA,
            pltpu.SemaphoreType.DMA,
            pltpu.SemaphoreType.DMA,
            pltpu.SemaphoreType.DMA,
        ],
    )(glo, ghi, src2, dst2)


BM = 1000
_GRID = N // BM
_bf16 = jnp.bfloat16


def _DOT(a, b):
    ah = a.astype(_bf16)
    ar = (a - ah.astype(_f32)).astype(_bf16)
    bh = b.astype(_bf16)
    br = (b - bh.astype(_f32)).astype(_bf16)
    d = functools.partial(jnp.dot, preferred_element_type=_f32)
    return d(ah, bh) + d(ah, br) + d(ar, bh)


def _mmprep_body(x_ref, w_ref, d0_ref, d1_ref, glo_ref, ghi_ref, dv_ref):
    dv = lax.rsqrt(d0_ref[...] + d1_ref[...] + 1.0)
    g = _DOT(x_ref[...], w_ref[...]) * dv
    glo_ref[...] = g[:, :DH]
    ghi_ref[...] = g[:, DH:]
    dv_ref[...] = dv


def _mid_body(alo_ref, ahi_ref, dv_ref, b1_ref, w2_ref, glo_ref, ghi_ref):
    a = jnp.concatenate([alo_ref[...], ahi_ref[...]], axis=1)
    z = jnp.maximum(a * dv_ref[...] + b1_ref[...], 0.0)
    h2 = _DOT(z, w2_ref[...])
    g2 = h2 * dv_ref[...]
    glo_ref[...] = g2[:, :DH]
    ghi_ref[...] = g2[:, DH:]


def _fin_body(alo_ref, ahi_ref, dv_ref, b2_ref, o_ref):
    a = jnp.concatenate([alo_ref[...], ahi_ref[...]], axis=1)
    o_ref[...] = a * dv_ref[...] + b2_ref[...]


def _row_spec(cols):
    return pl.BlockSpec((BM, cols), lambda i: (i, 0))


_FULL_W = pl.BlockSpec((D, D), lambda i: (0, 0))
_FULL_B = pl.BlockSpec((1, D), lambda i: (0, 0))
_DV = pl.BlockSpec((BM, 1), lambda i: (i, 0))


def _mmprep_call(x, W1, deg0c, deg1c):
    return pl.pallas_call(
        _mmprep_body,
        grid=(_GRID,),
        in_specs=[_row_spec(D), _FULL_W, _DV, _DV],
        out_specs=[_row_spec(DH), _row_spec(DH), _DV],
        out_shape=[jax.ShapeDtypeStruct((NP, DH), _f32),
                   jax.ShapeDtypeStruct((NP, DH), _f32),
                   jax.ShapeDtypeStruct((NP, 1), _f32)],
    )(x, W1, deg0c, deg1c)


def _mid_call(alo, ahi, dv, b1, W2):
    return pl.pallas_call(
        _mid_body,
        grid=(_GRID,),
        in_specs=[_row_spec(DH), _row_spec(DH), _DV, _FULL_B, _FULL_W],
        out_specs=[_row_spec(DH), _row_spec(DH)],
        out_shape=[jax.ShapeDtypeStruct((NP, DH), _f32),
                   jax.ShapeDtypeStruct((NP, DH), _f32)],
    )(alo, ahi, dv, b1, W2)


def _fin_call(alo, ahi, dv, b2):
    return pl.pallas_call(
        _fin_body,
        grid=(_GRID,),
        in_specs=[_row_spec(DH), _row_spec(DH), _DV, _FULL_B],
        out_specs=_row_spec(D),
        out_shape=jax.ShapeDtypeStruct((N, D), _f32),
    )(alo, ahi, dv, b2)


def kernel(src, dst, distances, x, edge_index, W1, b1, W2, b2):
    s_idx = src.astype(jnp.int32)
    d_idx = dst.astype(jnp.int32)

    pad_cyc = jnp.arange(PAD, dtype=jnp.int32) % (NP - N)
    src2 = jnp.concatenate([s_idx, pad_cyc]).reshape(EP // CH, CH)
    dst2 = jnp.concatenate([d_idx, N + pad_cyc]).reshape(EP // CH, CH)

    b1r = b1.reshape(1, D)
    b2r = b2.reshape(1, D)

    deg0, deg1 = _deg_call(dst2)
    glo1, ghi1, dv = _mmprep_call(x, W1, deg0.reshape(NP, 1),
                                  deg1.reshape(NP, 1))
    alo1, ahi1 = _agg_call(glo1, ghi1, src2, dst2)
    glo2, ghi2 = _mid_call(alo1, ahi1, dv, b1r, W2)
    alo2, ahi2 = _agg_call(glo2, ghi2, src2, dst2)
    return _fin_call(alo2, ahi2, dv, b2r)


# speedup vs baseline: 1.2238x; 1.0250x over previous
import functools

import jax
import jax.numpy as jnp
from jax import lax
from jax.experimental import pallas as pl
from jax.experimental.pallas import tpu as pltpu
from jax.experimental.pallas import tpu_sc as plsc

N = 10000
NP = 10240
D = 256
DH = 128
E = 160000
EP = 163840
PAD = EP - E
NSUB = 16
NCORE = 2
NPT = NP // NSUB
CH = 128
EPT = EP // NSUB
NCHUNK = EPT // CH
NPAIR = NCHUNK // 2
DEG_CW = 128
DEG_ROWS_W = EP // DEG_CW // (NCORE * NSUB)

_mesh = plsc.VectorSubcoreMesh(
    core_axis_name="c", subcore_axis_name="s",
    num_cores=NCORE, num_subcores=NSUB)

_f32 = jnp.float32


def _deg_body(dst2_hbm, deg0_hbm, deg1_hbm, dacc, idx_v, ones_v, zbuf):
    c = lax.axis_index("c")
    s = lax.axis_index("s")
    w = c * NSUB + s
    rsl = pl.ds(s * NPT, NPT)

    @pl.loop(0, NPT // 16)
    def _(i):
        zbuf[pl.ds(i * 16, 16)] = jnp.zeros((16,), _f32)

    @pl.loop(0, DEG_CW // 16)
    def _(i):
        ones_v[pl.ds(i * 16, 16)] = jnp.full((16,), 1.0, _f32)

    pltpu.sync_copy(zbuf, dacc.at[rsl])
    pltpu.sync_copy(dst2_hbm.at[pl.ds(w * DEG_ROWS_W, DEG_ROWS_W)], idx_v)
    plsc.subcore_barrier()

    @pl.loop(0, DEG_ROWS_W)
    def _(j):
        pltpu.sync_copy(ones_v, dacc.at[idx_v.at[j]], add=True)

    plsc.subcore_barrier()

    @pl.when(c == 0)
    def _():
        pltpu.sync_copy(dacc.at[rsl], deg0_hbm.at[rsl])

    @pl.when(c == 1)
    def _():
        pltpu.sync_copy(dacc.at[rsl], deg1_hbm.at[rsl])


def _deg_call(dst2):
    return pl.kernel(
        _deg_body,
        out_type=[jax.ShapeDtypeStruct((NP,), _f32),
                  jax.ShapeDtypeStruct((NP,), _f32)],
        mesh=_mesh,
        scratch_types=[
            pltpu.VMEM_SHARED((NP,), _f32),
            pltpu.VMEM((DEG_ROWS_W, DEG_CW), jnp.int32),
            pltpu.VMEM((DEG_CW,), _f32),
            pltpu.VMEM((NPT,), _f32),
        ],
    )(dst2)


def _agg_body(tlo_hbm, thi_hbm, src2_hbm, dst2_hbm, olo_hbm, ohi_hbm,
              acc, sA, dA, sB, dB, rows0, rows1,
              semiA, semiB, semg0, semg1):
    c = lax.axis_index("c")
    s = lax.axis_index("s")
    rsl = pl.ds(s * NPT, NPT)
    rbase = s * NCHUNK

    @pl.when(c == 0)
    def _():
        pltpu.sync_copy(tlo_hbm.at[rsl], acc.at[rsl])

    @pl.when(c == 1)
    def _():
        pltpu.sync_copy(thi_hbm.at[rsl], acc.at[rsl])

    def load_idx(p, sbuf, dbuf, sem):
        rows = pl.ds(rbase + 2 * p, 2)
        pltpu.async_copy(src2_hbm.at[rows], sbuf, sem)
        pltpu.async_copy(dst2_hbm.at[rows], dbuf, sem)

    def wait_idx(p, sbuf, dbuf, sem):
        rows = pl.ds(rbase + 2 * p, 2)
        pltpu.make_async_copy(src2_hbm.at[rows], sbuf, sem).wait()
        pltpu.make_async_copy(dst2_hbm.at[rows], dbuf, sem).wait()

    load_idx(0, sA, dA, semiA)
    load_idx(1, sB, dB, semiB)
    plsc.subcore_barrier()

    def run(tbl):
        def gather(sbuf, half, rows, sem):
            pltpu.async_copy(tbl.at[sbuf.at[half]], rows, sem)

        def gwait(rows, sem):
            pltpu.make_async_copy(tbl.at[sA.at[0]], rows, sem).wait()

        wait_idx(0, sA, dA, semiA)
        gather(sA, 0, rows0, semg0)
        gather(sA, 1, rows1, semg1)

        def pair_body(p, sS, dS, semiS, sT, dT, semiT):
            gwait(rows0, semg0)
            pltpu.sync_copy(rows0, acc.at[dS.at[0]], add=True)

            @pl.when(p + 1 < NPAIR)
            def _():
                wait_idx(p + 1, sT, dT, semiT)
                gather(sT, 0, rows0, semg0)

            gwait(rows1, semg1)
            pltpu.sync_copy(rows1, acc.at[dS.at[1]], add=True)

            @pl.when(p + 2 < NPAIR)
            def _():
                load_idx(p + 2, sS, dS, semiS)

            @pl.when(p + 1 < NPAIR)
            def _():
                gather(sT, 1, rows1, semg1)

        @pl.loop(0, NPAIR // 2)
        def _(q):
            pair_body(2 * q, sA, dA, semiA, sB, dB, semiB)
            pair_body(2 * q + 1, sB, dB, semiB, sA, dA, semiA)

    @pl.when(c == 0)
    def _():
        run(tlo_hbm)

    @pl.when(c == 1)
    def _():
        run(thi_hbm)

    plsc.subcore_barrier()

    @pl.when(c == 0)
    def _():
        pltpu.sync_copy(acc.at[rsl], olo_hbm.at[rsl])

    @pl.when(c == 1)
    def _():
        pltpu.sync_copy(acc.at[rsl], ohi_hbm.at[rsl])


def _agg_call(glo, ghi, src2, dst2):
    return pl.kernel(
        _agg_body,
        out_type=[jax.ShapeDtypeStruct((NP, DH), _f32),
                  jax.ShapeDtypeStruct((NP, DH), _f32)],
        mesh=_mesh,
        scratch_types=[
            pltpu.VMEM_SHARED((NP, DH), _f32),
            pltpu.VMEM((2, CH), jnp.int32),
            pltpu.VMEM((2, CH), jnp.int32),
            pltpu.VMEM((2, CH), jnp.int32),
            pltpu.VMEM((2, CH), jnp.int32),
            pltpu.VMEM((CH, DH), _f32),
            pltpu.VMEM((CH, DH), _f32),
            pltpu.SemaphoreType.DMA,
            pltpu.SemaphoreType.DMA,
            pltpu.SemaphoreType.DMA,
            pltpu.SemaphoreType.DMA,
        ],
    )(glo, ghi, src2, dst2)


BM = 2000
_GRID = N // BM
_bf16 = jnp.bfloat16


def _DOT(a, b):
    ah = a.astype(_bf16)
    ar = (a - ah.astype(_f32)).astype(_bf16)
    bh = b.astype(_bf16)
    br = (b - bh.astype(_f32)).astype(_bf16)
    d = functools.partial(jnp.dot, preferred_element_type=_f32)
    return d(ah, bh) + d(ah, br) + d(ar, bh)


def _mmprep_body(x_ref, w_ref, d0_ref, d1_ref, glo_ref, ghi_ref, dv_ref):
    dv = lax.rsqrt(d0_ref[...] + d1_ref[...] + 1.0)
    g = _DOT(x_ref[...], w_ref[...]) * dv
    glo_ref[...] = g[:, :DH]
    ghi_ref[...] = g[:, DH:]
    dv_ref[...] = dv


def _mid_body(alo_ref, ahi_ref, dv_ref, b1_ref, w2_ref, glo_ref, ghi_ref):
    a = jnp.concatenate([alo_ref[...], ahi_ref[...]], axis=1)
    z = jnp.maximum(a * dv_ref[...] + b1_ref[...], 0.0)
    h2 = _DOT(z, w2_ref[...])
    g2 = h2 * dv_ref[...]
    glo_ref[...] = g2[:, :DH]
    ghi_ref[...] = g2[:, DH:]


def _fin_body(alo_ref, ahi_ref, dv_ref, b2_ref, o_ref):
    a = jnp.concatenate([alo_ref[...], ahi_ref[...]], axis=1)
    o_ref[...] = a * dv_ref[...] + b2_ref[...]


def _row_spec(cols):
    return pl.BlockSpec((BM, cols), lambda i: (i, 0))


_FULL_W = pl.BlockSpec((D, D), lambda i: (0, 0))
_FULL_B = pl.BlockSpec((1, D), lambda i: (0, 0))
_DV = pl.BlockSpec((BM, 1), lambda i: (i, 0))


def _mmprep_call(x, W1, deg0c, deg1c):
    return pl.pallas_call(
        _mmprep_body,
        grid=(_GRID,),
        in_specs=[_row_spec(D), _FULL_W, _DV, _DV],
        out_specs=[_row_spec(DH), _row_spec(DH), _DV],
        out_shape=[jax.ShapeDtypeStruct((NP, DH), _f32),
                   jax.ShapeDtypeStruct((NP, DH), _f32),
                   jax.ShapeDtypeStruct((NP, 1), _f32)],
    )(x, W1, deg0c, deg1c)


def _mid_call(alo, ahi, dv, b1, W2):
    return pl.pallas_call(
        _mid_body,
        grid=(_GRID,),
        in_specs=[_row_spec(DH), _row_spec(DH), _DV, _FULL_B, _FULL_W],
        out_specs=[_row_spec(DH), _row_spec(DH)],
        out_shape=[jax.ShapeDtypeStruct((NP, DH), _f32),
                   jax.ShapeDtypeStruct((NP, DH), _f32)],
    )(alo, ahi, dv, b1, W2)


def _fin_call(alo, ahi, dv, b2):
    return pl.pallas_call(
        _fin_body,
        grid=(_GRID,),
        in_specs=[_row_spec(DH), _row_spec(DH), _DV, _FULL_B],
        out_specs=_row_spec(D),
        out_shape=jax.ShapeDtypeStruct((N, D), _f32),
    )(alo, ahi, dv, b2)


def kernel(src, dst, distances, x, edge_index, W1, b1, W2, b2):
    s_idx = src.astype(jnp.int32)
    d_idx = dst.astype(jnp.int32)

    pad_cyc = jnp.arange(PAD, dtype=jnp.int32) % (NP - N)
    src2 = jnp.concatenate([s_idx, pad_cyc]).reshape(EP // CH, CH)
    dst2 = jnp.concatenate([d_idx, N + pad_cyc]).reshape(EP // CH, CH)

    b1r = b1.reshape(1, D)
    b2r = b2.reshape(1, D)

    deg0, deg1 = _deg_call(dst2)
    glo1, ghi1, dv = _mmprep_call(x, W1, deg0.reshape(NP, 1),
                                  deg1.reshape(NP, 1))
    alo1, ahi1 = _agg_call(glo1, ghi1, src2, dst2)
    glo2, ghi2 = _mid_call(alo1, ahi1, dv, b1r, W2)
    alo2, ahi2 = _agg_call(glo2, ghi2, src2, dst2)
    return _fin_call(alo2, ahi2, dv, b2r)
